# Initial kernel scaffold; baseline (speedup 1.0000x reference)
import jax
import jax.numpy as jnp
from jax.experimental import pallas as pl


def kernel(node_states, edges, W_e, b_e, W_u, b_u):
    raise NotImplementedError("write your pallas kernel here")


# speedup vs baseline: 6.2741x; 6.2741x over previous
import functools

import jax
import jax.numpy as jnp
from jax import lax
from jax.experimental import pallas as pl
from jax.experimental.pallas import tpu as pltpu
from jax.experimental.pallas import tpu_sc as plsc

N = 10000
E = 320000
D = 128

NC = 2
NS = 16
L = 16
NW = NC * NS

C = 128
NCHUNK = E // C
ROWS_PER_TILE = 624
TAIL_BASE = NS * ROWS_PER_TILE
TAIL_ROWS = N - TAIL_BASE

ROW_BLK = 400


def _pq_body(ns_ref, wsrc_ref, wdst_ref, be_ref, p_ref, q_ref):
    ns = ns_ref[...]
    p_ref[...] = jnp.dot(ns, wsrc_ref[...], preferred_element_type=jnp.float32)
    q_ref[...] = (
        jnp.dot(ns, wdst_ref[...], preferred_element_type=jnp.float32)
        + be_ref[...]
    )


def _pq(node_states, w_src, w_dst, b_e_row):
    return pl.pallas_call(
        _pq_body,
        grid=(N // ROW_BLK,),
        in_specs=[
            pl.BlockSpec((ROW_BLK, D), lambda i: (i, 0)),
            pl.BlockSpec((D, D), lambda i: (0, 0)),
            pl.BlockSpec((D, D), lambda i: (0, 0)),
            pl.BlockSpec((1, D), lambda i: (0, 0)),
        ],
        out_specs=[
            pl.BlockSpec((ROW_BLK, D), lambda i: (i, 0)),
            pl.BlockSpec((ROW_BLK, D), lambda i: (i, 0)),
        ],
        out_shape=[
            jax.ShapeDtypeStruct((N, D), jnp.float32),
            jax.ShapeDtypeStruct((N, D), jnp.float32),
        ],
    )(node_states, w_src, w_dst, b_e_row)


def _upd_body(ns_ref, pp_ref, wt_ref, wb_ref, bu_ref, o_ref):
    aggr = pp_ref[0] + pp_ref[1]
    acc = jnp.dot(ns_ref[...], wt_ref[...], preferred_element_type=jnp.float32)
    acc = acc + jnp.dot(aggr, wb_ref[...], preferred_element_type=jnp.float32)
    o_ref[...] = jnp.maximum(acc + bu_ref[...], 0.0)


def _upd(node_states, partials, w_top, w_bot, b_u_row):
    return pl.pallas_call(
        _upd_body,
        grid=(N // ROW_BLK,),
        in_specs=[
            pl.BlockSpec((ROW_BLK, D), lambda i: (i, 0)),
            pl.BlockSpec((NC, ROW_BLK, D), lambda i: (0, i, 0)),
            pl.BlockSpec((D, D), lambda i: (0, 0)),
            pl.BlockSpec((D, D), lambda i: (0, 0)),
            pl.BlockSpec((1, D), lambda i: (0, 0)),
        ],
        out_specs=pl.BlockSpec((ROW_BLK, D), lambda i: (i, 0)),
        out_shape=jax.ShapeDtypeStruct((N, D), jnp.float32),
    )(node_states, partials, w_top, w_bot, b_u_row)


def _sc_edge_body(p_hbm, q_hbm, src_hbm, dst_hbm, out_hbm,
                  sidx, didx, arows, brows, aggr, sem1, sem2):
    cid = lax.axis_index("c")
    sid = lax.axis_index("s")
    wid = cid * NS + sid

    @pl.loop(0, C)
    def _zero_rows(r):
        @pl.loop(0, D, step=L)
        def _zero_cols(c0):
            arows[r, pl.ds(c0, L)] = jnp.zeros((L,), jnp.float32)

    zbase = sid * ROWS_PER_TILE
    @pl.loop(0, 4)
    def _zcopy(k):
        pltpu.sync_copy(arows, aggr.at[pl.ds(zbase + k * C, C)])
    pltpu.sync_copy(arows.at[pl.ds(0, ROWS_PER_TILE - 4 * C)],
                    aggr.at[pl.ds(zbase + 4 * C, ROWS_PER_TILE - 4 * C)])

    @pl.when(sid == NS - 1)
    def _ztail():
        pltpu.sync_copy(arows.at[pl.ds(0, TAIL_ROWS)],
                        aggr.at[pl.ds(TAIL_BASE, TAIL_ROWS)])

    plsc.subcore_barrier()

    @pl.loop(wid, NCHUNK, step=NW)
    def _chunk(g):
        ebase = g * C
        pltpu.sync_copy(src_hbm.at[pl.ds(ebase, C)], sidx)
        pltpu.sync_copy(dst_hbm.at[pl.ds(ebase, C)], didx.at[0])
        cp1 = pltpu.async_copy(p_hbm.at[sidx], arows, sem1)
        cp2 = pltpu.async_copy(q_hbm.at[didx.at[0]], brows, sem2)
        cp1.wait()
        cp2.wait()

        @pl.loop(0, C)
        def _row(r):
            @pl.loop(0, D, step=L)
            def _col(c0):
                s = pl.ds(c0, L)
                arows[r, s] = jnp.maximum(arows[r, s] + brows[r, s], 0.0)

        pltpu.sync_copy(arows, aggr.at[didx.at[0]], add=True)

    plsc.subcore_barrier()
    pltpu.sync_copy(aggr.at[pl.ds(zbase, ROWS_PER_TILE)],
                    out_hbm.at[cid].at[pl.ds(zbase, ROWS_PER_TILE)])

    @pl.when(sid == NS - 1)
    def _otail():
        pltpu.sync_copy(aggr.at[pl.ds(TAIL_BASE, TAIL_ROWS)],
                        out_hbm.at[cid].at[pl.ds(TAIL_BASE, TAIL_ROWS)])


@jax.jit
def _sc_edge(p, q, src, dst):
    mesh = plsc.VectorSubcoreMesh(
        core_axis_name="c", subcore_axis_name="s",
        num_cores=NC, num_subcores=NS)
    k = pl.kernel(
        _sc_edge_body,
        out_type=jax.ShapeDtypeStruct((NC, N, D), jnp.float32),
        mesh=mesh,
        scratch_types=[
            pltpu.VMEM((C,), jnp.int32),
            pltpu.VMEM((1, C), jnp.int32),
            pltpu.VMEM((C, D), jnp.float32),
            pltpu.VMEM((C, D), jnp.float32),
            pltpu.VMEM_SHARED((N, D), jnp.float32),
            pltpu.SemaphoreType.DMA,
            pltpu.SemaphoreType.DMA,
        ],
    )
    return k(p, q, src, dst)


def kernel(node_states, edges, W_e, b_e, W_u, b_u):
    src = edges[:, 0]
    dst = edges[:, 1]
    p, q = _pq(node_states, W_e[:D], W_e[D:], b_e.reshape(1, D))
    partials = _sc_edge(p, q, src, dst)
    return _upd(node_states, partials, W_u[:D], W_u[D:], b_u.reshape(1, D))
